# baseline (speedup 1.0000x reference)
import functools

import jax
import jax.numpy as jnp
import numpy as np
from jax import lax
from jax.experimental import pallas as pl
from jax.experimental.pallas import tpu as pltpu
from jax.experimental.pallas import tpu_sc as plsc

_N = 10000
_E = 160000
_D = 384
_DH = 192
_NB = 10
_CUT = 4.0
_L = 3
_NSP = 119
_NNEI = 16.0
_SH = 9

_c64 = np.linspace(0.0, _CUT, _NB)
_WIDTH = float(_c64[1] - _c64[0])
_rs = np.linspace(0.0, _CUT, 4001)[1:]
_bs = np.exp(-(((_rs[:, None] - _c64[None, :]) / _WIDTH) ** 2)) * 1.12
_RBF_CONSTS = np.stack([
    _c64.astype(np.float32),
    _bs.mean(axis=0).astype(np.float32),
    (1.0 / (_bs.std(axis=0) + 1e-9)).astype(np.float32),
])


def _edges_body(ps_ref, pd_ref, ed_ref, cell_ref, fc1_ref, fc2_ref, S_ref, rbf_ref,
                g0l, g0h, g1l, g1h, g2l, g2h):
    ps = ps_ref[...]
    pd = pd_ref[...]
    ed = ed_ref[...]
    cell = cell_ref[...]
    disp = jnp.dot(ed, cell, preferred_element_type=jnp.float32)
    vec = pd[:, :3] - ps[:, :3] - disp
    r = jnp.sqrt(jnp.sum(vec * vec, axis=1, keepdims=True))
    u = vec / (r + 1e-9)
    x, y, z = u[:, 0:1], u[:, 1:2], u[:, 2:3]
    s3 = 3.0 ** 0.5
    s15 = 15.0 ** 0.5
    s5 = 5.0 ** 0.5
    sh = jnp.concatenate([
        jnp.ones_like(x), s3 * x, s3 * y, s3 * z,
        s15 * x * y, s15 * y * z, (s5 / 2.0) * (3.0 * z * z - 1.0),
        s15 * x * z, (s15 / 2.0) * (x * x - y * y)], axis=1)
    centers = rbf_ref[0:1, :]
    rmean = rbf_ref[1:2, :]
    ristd = rbf_ref[2:3, :]
    es = (jnp.exp(-(((r - centers) / _WIDTH) ** 2)) * 1.12 - rmean) * ristd
    outs = ((g0l, g0h), (g1l, g1h), (g2l, g2h))
    for l in range(_L):
        w = jax.nn.silu(jnp.dot(es, fc1_ref[l], preferred_element_type=jnp.float32))
        w = jnp.dot(w, fc2_ref[l], preferred_element_type=jnp.float32)
        gate = jnp.dot(w * sh, S_ref[l], preferred_element_type=jnp.float32)
        outs[l][0][...] = gate[:, :_DH]
        outs[l][1][...] = gate[:, _DH:]


def _emb_body(nodes_ref, emb_ref, x_ref):
    nd = nodes_ref[...]
    ids = lax.broadcasted_iota(jnp.int32, (1, 128), 1)
    oh = (nd == ids).astype(jnp.float32)
    x_ref[...] = jnp.dot(oh, emb_ref[...], preferred_element_type=jnp.float32)


def _pre_body(x_ref, wsc_ref, wl1_ref, nsc_ref, nf_ref):
    x = x_ref[...]
    nsc_ref[...] = jnp.dot(x, wsc_ref[...], preferred_element_type=jnp.float32)
    nf_ref[...] = jnp.dot(x, wl1_ref[...], preferred_element_type=jnp.float32)


def _post_body(al_ref, ah_ref, nsc_ref, wl2_ref, wl3_ref, x_ref):
    agg = jnp.concatenate([al_ref[...], ah_ref[...]], axis=1)
    agg = agg * np.float32(1.0 / np.sqrt(_NNEI))
    conv = jnp.dot(agg, wl2_ref[...], preferred_element_type=jnp.float32)
    ang = 0.1 * jnp.dot(agg, wl3_ref[...], preferred_element_type=jnp.float32)
    x_ref[...] = jnp.cos(ang) * nsc_ref[...] + jnp.sin(ang) * conv


def kernel(nodes, positions, cells, edges, edges_displacement, splits, Emb,
           Wsc, Wlin1, Wlin2, Wlin3, fc1, fc2, S):
    f32 = jnp.float32
    src = edges[:, 0].astype(jnp.int32)
    dst = edges[:, 1].astype(jnp.int32)
    pos16 = jnp.zeros((_N, 16), f32).at[:, :3].set(positions)
    cell = cells[0]
    emb_pad = jnp.zeros((128, _D), f32).at[:_NSP].set(Emb)
    nodes2 = nodes.astype(jnp.int32)[:, None]

    ps16 = pos16[src]
    pd16 = pos16[dst]

    BE = 2000
    gates = pl.pallas_call(
        _edges_body,
        grid=(_E // BE,),
        in_specs=[
            pl.BlockSpec((BE, 16), lambda i: (i, 0)),
            pl.BlockSpec((BE, 16), lambda i: (i, 0)),
            pl.BlockSpec((BE, 3), lambda i: (i, 0)),
            pl.BlockSpec((3, 3), lambda i: (0, 0)),
            pl.BlockSpec((_L, _NB, 100), lambda i: (0, 0, 0)),
            pl.BlockSpec((_L, 100, _SH), lambda i: (0, 0, 0)),
            pl.BlockSpec((_L, _SH, _D), lambda i: (0, 0, 0)),
            pl.BlockSpec((3, _NB), lambda i: (0, 0)),
        ],
        out_specs=[pl.BlockSpec((BE, _DH), lambda i: (i, 0))] * 6,
        out_shape=[jax.ShapeDtypeStruct((_E, _DH), f32)] * 6,
    )(ps16, pd16, edges_displacement, cell, fc1, fc2, S,
      jnp.asarray(_RBF_CONSTS))

    BN = 2000
    xfeat = pl.pallas_call(
        _emb_body,
        grid=(_N // BN,),
        in_specs=[
            pl.BlockSpec((BN, 1), lambda i: (i, 0)),
            pl.BlockSpec((128, _D), lambda i: (0, 0)),
        ],
        out_specs=pl.BlockSpec((BN, _D), lambda i: (i, 0)),
        out_shape=jax.ShapeDtypeStruct((_N, _D), f32),
    )(nodes2, emb_pad)

    for l in range(_L):
        nsc, nf = pl.pallas_call(
            _pre_body,
            grid=(_N // BN,),
            in_specs=[
                pl.BlockSpec((BN, _D), lambda i: (i, 0)),
                pl.BlockSpec((_D, _D), lambda i: (0, 0)),
                pl.BlockSpec((_D, _D), lambda i: (0, 0)),
            ],
            out_specs=[pl.BlockSpec((BN, _D), lambda i: (i, 0))] * 2,
            out_shape=[jax.ShapeDtypeStruct((_N, _D), f32)] * 2,
        )(xfeat, Wsc[l], Wlin1[l])

        gate = jnp.concatenate([gates[2 * l], gates[2 * l + 1]], axis=1)
        ef = nf[src] * gate
        agg = jax.ops.segment_sum(ef, dst, num_segments=_N)
        agg_lo = agg[:, :_DH]
        agg_hi = agg[:, _DH:]

        xfeat = pl.pallas_call(
            _post_body,
            grid=(_N // BN,),
            in_specs=[
                pl.BlockSpec((BN, _DH), lambda i: (i, 0)),
                pl.BlockSpec((BN, _DH), lambda i: (i, 0)),
                pl.BlockSpec((BN, _D), lambda i: (i, 0)),
                pl.BlockSpec((_D, _D), lambda i: (0, 0)),
                pl.BlockSpec((_D, 1), lambda i: (0, 0)),
            ],
            out_specs=pl.BlockSpec((BN, _D), lambda i: (i, 0)),
            out_shape=jax.ShapeDtypeStruct((_N, _D), f32),
        )(agg_lo, agg_hi, nsc, Wlin2[l], Wlin3[l])

    return xfeat


# speedup vs baseline: 1.7534x; 1.7534x over previous
import functools

import jax
import jax.numpy as jnp
import numpy as np
from jax import lax
from jax.experimental import pallas as pl
from jax.experimental.pallas import tpu as pltpu
from jax.experimental.pallas import tpu_sc as plsc

_N = 10000
_E = 160000
_EP = 160016
_D = 384
_DC = 128
_NB = 10
_CUT = 4.0
_L = 3
_NSP = 119
_NNEI = 16.0
_SH = 9

_c64 = np.linspace(0.0, _CUT, _NB)
_WIDTH = float(_c64[1] - _c64[0])
_rs = np.linspace(0.0, _CUT, 4001)[1:]
_bs = np.exp(-(((_rs[:, None] - _c64[None, :]) / _WIDTH) ** 2)) * 1.12
_RBF_CONSTS = np.stack([
    _c64.astype(np.float32),
    _bs.mean(axis=0).astype(np.float32),
    (1.0 / (_bs.std(axis=0) + 1e-9)).astype(np.float32),
])

_SC_MESH = plsc.VectorSubcoreMesh(core_axis_name="c", subcore_axis_name="s")
_NW = 32
_EPW = _E // _NW
_NCH = _EPW // 16 + 1
_BA = 80
_BB = 40
_ZR = 80
_NZ = _N // _ZR


@functools.partial(
    pl.kernel,
    out_type=jax.ShapeDtypeStruct((_EP * 8,), jnp.float32),
    mesh=_SC_MESH,
    scratch_types=[
        pltpu.VMEM((3 * _N,), jnp.float32),
        pltpu.VMEM((_EPW + 16,), jnp.int32),
        pltpu.VMEM((_EPW + 16,), jnp.int32),
        pltpu.VMEM(((_EPW + 16) * 8,), jnp.float32),
        pltpu.SemaphoreType.DMA,
    ],
    compiler_params=pltpu.CompilerParams(needs_layout_passes=False),
)
def _pos_gather(pos_hbm, src_hbm, dst_hbm, geo_hbm,
                pos_v, sidx_v, didx_v, out_v, sem):
    c = lax.axis_index("c")
    s = lax.axis_index("s")
    w = s * 2 + c
    e0 = pl.multiple_of(w * _EPW, 8)
    pltpu.sync_copy(pos_hbm, pos_v)
    pltpu.sync_copy(src_hbm.at[pl.ds(e0, _EPW + 16)], sidx_v)
    pltpu.sync_copy(dst_hbm.at[pl.ds(e0, _EPW + 16)], didx_v)

    def body(i, carry):
        e8 = (lax.broadcasted_iota(jnp.int32, (16,), 0) + i * 16) * 8
        si3 = sidx_v[pl.ds(i * 16, 16)] * 3
        di3 = didx_v[pl.ds(i * 16, 16)] * 3
        for cmp in range(3):
            plsc.store_scatter(out_v, [e8 + cmp],
                               plsc.load_gather(pos_v, [si3 + cmp]))
            plsc.store_scatter(out_v, [e8 + (cmp + 3)],
                               plsc.load_gather(pos_v, [di3 + cmp]))
        return carry

    lax.fori_loop(0, _NCH, body, 0)
    pltpu.sync_copy(out_v, geo_hbm.at[pl.ds(e0 * 8, (_EPW + 16) * 8)])


@functools.partial(
    pl.kernel,
    out_type=[jax.ShapeDtypeStruct((2, _N, _DC), jnp.float32),
              jax.ShapeDtypeStruct((2, _N, _DC), jnp.float32)],
    mesh=_SC_MESH,
    scratch_types=[
        pltpu.VMEM((_BA,), jnp.int32),
        pltpu.VMEM((_BA,), jnp.int32),
        pltpu.VMEM((_BA, _DC), jnp.float32),
        pltpu.VMEM((_BA, _DC), jnp.float32),
        pltpu.VMEM((_BB,), jnp.int32),
        pltpu.VMEM((_BB,), jnp.int32),
        pltpu.VMEM((_ZR, _DC), jnp.float32),
        pltpu.VMEM_SHARED((_N, _DC), jnp.float32),
        pltpu.SemaphoreType.DMA,
    ],
)
def _msgpass(nf0_hbm, nf1_hbm, nf2_hbm, gl_hbm, src_hbm, dst_hbm,
             aggA_hbm, aggB_hbm,
             sa_v, da_v, rowsa_v, gatea_v, sb_v, db_v,
             zb_v, acc_sh, sem):
    c = lax.axis_index("c")
    s = lax.axis_index("s")

    def zero_acc():
        def zrow(i, carry):
            zero = jnp.zeros((16,), jnp.float32)
            for j in range(_DC // 16):
                zb_v[i, pl.ds(j * 16, 16)] = zero
            return carry

        lax.fori_loop(0, _ZR, zrow, 0)
        for k in range(8):
            idx = s + k * 16

            @pl.when(idx < _NZ)
            def _():
                r0 = pl.multiple_of(idx * _ZR, 8)
                pltpu.sync_copy(zb_v, acc_sh.at[pl.ds(r0, _ZR)])

    def dump_acc(out_hbm):
        for k in range(8):
            idx = s + k * 16

            @pl.when(idx < _NZ)
            def _():
                r0 = pl.multiple_of(idx * _ZR, 8)
                pltpu.sync_copy(acc_sh.at[pl.ds(r0, _ZR)], zb_v)
                pltpu.sync_copy(zb_v, out_hbm.at[c, pl.ds(r0, _ZR)])

    zero_acc()
    plsc.subcore_barrier()

    def body_a(i, carry):
        e0 = pl.multiple_of(s * (_E // 16) + i * _BA, 8)
        pltpu.sync_copy(src_hbm.at[pl.ds(e0, _BA)], sa_v)
        pltpu.sync_copy(dst_hbm.at[pl.ds(e0, _BA)], da_v)

        @pl.when(c == 0)
        def _():
            pltpu.async_copy(nf0_hbm.at[sa_v], rowsa_v, sem).wait()

        @pl.when(c == 1)
        def _():
            pltpu.async_copy(nf1_hbm.at[sa_v], rowsa_v, sem).wait()

        pltpu.sync_copy(gl_hbm.at[c, pl.ds(e0, _BA)], gatea_v)

        def mul(ii, cc):
            for j in range(_DC // 16):
                rowsa_v[ii, pl.ds(j * 16, 16)] = (
                    rowsa_v[ii, pl.ds(j * 16, 16)]
                    * gatea_v[ii, pl.ds(j * 16, 16)])
            return cc

        lax.fori_loop(0, _BA, mul, 0)
        pltpu.sync_copy(rowsa_v, acc_sh.at[da_v], add=True)
        return carry

    lax.fori_loop(0, (_E // 16) // _BA, body_a, 0)
    plsc.subcore_barrier()
    dump_acc(aggA_hbm)
    plsc.subcore_barrier()

    zero_acc()
    plsc.subcore_barrier()

    def body_b(i, carry):
        e0 = pl.multiple_of(c * (_E // 2) + s * (_E // 32) + i * _BB, 8)
        pltpu.sync_copy(src_hbm.at[pl.ds(e0, _BB)], sb_v)
        pltpu.sync_copy(dst_hbm.at[pl.ds(e0, _BB)], db_v)
        pltpu.async_copy(nf2_hbm.at[sb_v], rowsa_v.at[pl.ds(0, _BB)], sem).wait()
        pltpu.sync_copy(gl_hbm.at[2, pl.ds(e0, _BB)], gatea_v.at[pl.ds(0, _BB)])

        def mul(ii, cc):
            for j in range(_DC // 16):
                rowsa_v[ii, pl.ds(j * 16, 16)] = (
                    rowsa_v[ii, pl.ds(j * 16, 16)]
                    * gatea_v[ii, pl.ds(j * 16, 16)])
            return cc

        lax.fori_loop(0, _BB, mul, 0)
        pltpu.sync_copy(rowsa_v.at[pl.ds(0, _BB)], acc_sh.at[db_v], add=True)
        return carry

    lax.fori_loop(0, (_E // 32) // _BB, body_b, 0)
    plsc.subcore_barrier()
    dump_acc(aggB_hbm)


def _edges_body(geo_ref, ed_ref, cell_ref, fc1_ref, fc2_ref, S_ref, rbf_ref,
                g0_ref, g1_ref, g2_ref):
    geo = geo_ref[...]
    ed = ed_ref[...]
    cell = cell_ref[...]
    disp = jnp.dot(ed, cell, preferred_element_type=jnp.float32)
    vec = geo[:, 3:6] - geo[:, 0:3] - disp
    r = jnp.sqrt(jnp.sum(vec * vec, axis=1, keepdims=True))
    u = vec / (r + 1e-9)
    x, y, z = u[:, 0:1], u[:, 1:2], u[:, 2:3]
    s3 = 3.0 ** 0.5
    s15 = 15.0 ** 0.5
    s5 = 5.0 ** 0.5
    sh = jnp.concatenate([
        jnp.ones_like(x), s3 * x, s3 * y, s3 * z,
        s15 * x * y, s15 * y * z, (s5 / 2.0) * (3.0 * z * z - 1.0),
        s15 * x * z, (s15 / 2.0) * (x * x - y * y)], axis=1)
    centers = rbf_ref[0:1, :]
    rmean = rbf_ref[1:2, :]
    ristd = rbf_ref[2:3, :]
    es = (jnp.exp(-(((r - centers) / _WIDTH) ** 2)) * 1.12 - rmean) * ristd
    for l, g_ref in enumerate((g0_ref, g1_ref, g2_ref)):
        w = jax.nn.silu(jnp.dot(es, fc1_ref[l], preferred_element_type=jnp.float32))
        w = jnp.dot(w, fc2_ref[l], preferred_element_type=jnp.float32)
        gate = jnp.dot(w * sh, S_ref[l], preferred_element_type=jnp.float32)
        g_ref[0] = gate[:, :_DC]
        g_ref[1] = gate[:, _DC:2 * _DC]
        g_ref[2] = gate[:, 2 * _DC:]


def _emb_body(nodes_ref, emb_ref, x_ref):
    nd = nodes_ref[...]
    ids = lax.broadcasted_iota(jnp.int32, (1, 128), 1)
    oh = (nd == ids).astype(jnp.float32)
    x_ref[...] = jnp.dot(oh, emb_ref[...], preferred_element_type=jnp.float32)


def _pre_body(x_ref, wsc_ref, wl1_ref, nsc_ref, nf0_ref, nf1_ref, nf2_ref):
    x = x_ref[...]
    nsc_ref[...] = jnp.dot(x, wsc_ref[...], preferred_element_type=jnp.float32)
    nf = jnp.dot(x, wl1_ref[...], preferred_element_type=jnp.float32)
    nf0_ref[...] = nf[:, :_DC]
    nf1_ref[...] = nf[:, _DC:2 * _DC]
    nf2_ref[...] = nf[:, 2 * _DC:]


def _post_body(aggA_ref, aggB_ref, nsc_ref, wl2_ref, wl3_ref, x_ref):
    agg = jnp.concatenate(
        [aggA_ref[0], aggA_ref[1], aggB_ref[0] + aggB_ref[1]], axis=1)
    agg = agg * np.float32(1.0 / np.sqrt(_NNEI))
    conv = jnp.dot(agg, wl2_ref[...], preferred_element_type=jnp.float32)
    ang = 0.1 * jnp.dot(agg, wl3_ref[...], preferred_element_type=jnp.float32)
    x_ref[...] = jnp.cos(ang) * nsc_ref[...] + jnp.sin(ang) * conv


def kernel(nodes, positions, cells, edges, edges_displacement, splits, Emb,
           Wsc, Wlin1, Wlin2, Wlin3, fc1, fc2, S):
    f32 = jnp.float32
    src = edges[:, 0].astype(jnp.int32)
    dst = edges[:, 1].astype(jnp.int32)
    srcp = jnp.concatenate([src, jnp.zeros((16,), jnp.int32)])
    dstp = jnp.concatenate([dst, jnp.zeros((16,), jnp.int32)])
    posflat = positions.reshape(-1)
    cell = cells[0]
    emb_pad = jnp.zeros((128, _D), f32).at[:_NSP].set(Emb)
    nodes2 = nodes.astype(jnp.int32)[:, None]

    geo = _pos_gather(posflat, srcp, dstp).reshape(_EP, 8)

    BE = 2000
    gates = pl.pallas_call(
        _edges_body,
        grid=(_E // BE,),
        in_specs=[
            pl.BlockSpec((BE, 8), lambda i: (i, 0)),
            pl.BlockSpec((BE, 3), lambda i: (i, 0)),
            pl.BlockSpec((3, 3), lambda i: (0, 0)),
            pl.BlockSpec((_L, _NB, 100), lambda i: (0, 0, 0)),
            pl.BlockSpec((_L, 100, _SH), lambda i: (0, 0, 0)),
            pl.BlockSpec((_L, _SH, _D), lambda i: (0, 0, 0)),
            pl.BlockSpec((3, _NB), lambda i: (0, 0)),
        ],
        out_specs=[pl.BlockSpec((3, BE, _DC), lambda i: (0, i, 0))] * 3,
        out_shape=[jax.ShapeDtypeStruct((3, _E, _DC), f32)] * 3,
    )(geo, edges_displacement, cell, fc1, fc2, S, jnp.asarray(_RBF_CONSTS))

    BN = 2000
    xfeat = pl.pallas_call(
        _emb_body,
        grid=(_N // BN,),
        in_specs=[
            pl.BlockSpec((BN, 1), lambda i: (i, 0)),
            pl.BlockSpec((128, _D), lambda i: (0, 0)),
        ],
        out_specs=pl.BlockSpec((BN, _D), lambda i: (i, 0)),
        out_shape=jax.ShapeDtypeStruct((_N, _D), f32),
    )(nodes2, emb_pad)

    for l in range(_L):
        nsc, nf0, nf1, nf2 = pl.pallas_call(
            _pre_body,
            grid=(_N // BN,),
            in_specs=[
                pl.BlockSpec((BN, _D), lambda i: (i, 0)),
                pl.BlockSpec((_D, _D), lambda i: (0, 0)),
                pl.BlockSpec((_D, _D), lambda i: (0, 0)),
            ],
            out_specs=[pl.BlockSpec((BN, _D), lambda i: (i, 0))]
            + [pl.BlockSpec((BN, _DC), lambda i: (i, 0))] * 3,
            out_shape=[jax.ShapeDtypeStruct((_N, _D), f32)]
            + [jax.ShapeDtypeStruct((_N, _DC), f32)] * 3,
        )(xfeat, Wsc[l], Wlin1[l])

        aggA, aggB = _msgpass(nf0, nf1, nf2, gates[l], src, dst)

        xfeat = pl.pallas_call(
            _post_body,
            grid=(_N // BN,),
            in_specs=[
                pl.BlockSpec((2, BN, _DC), lambda i: (0, i, 0)),
                pl.BlockSpec((2, BN, _DC), lambda i: (0, i, 0)),
                pl.BlockSpec((BN, _D), lambda i: (i, 0)),
                pl.BlockSpec((_D, _D), lambda i: (0, 0)),
                pl.BlockSpec((_D, 1), lambda i: (0, 0)),
            ],
            out_specs=pl.BlockSpec((BN, _D), lambda i: (i, 0)),
            out_shape=jax.ShapeDtypeStruct((_N, _D), f32),
        )(aggA, aggB, nsc, Wlin2[l], Wlin3[l])

    return xfeat
